# baseline (speedup 1.0000x reference)
import functools

import jax
import jax.numpy as jnp
from jax import lax
from jax.experimental import pallas as pl
from jax.experimental.pallas import tpu as pltpu
from jax.experimental.pallas import tpu_sc as plsc

_NC = 2
_NS = 16
_NW = _NC * _NS


def kernel(k_cache, v_cache, xk, xv, layer_past_len):
    B, Hkv, S, D = k_cache.shape
    N = B * Hkv
    slabs_per_w = N // _NW
    rows_per_core = N // _NC
    k2 = k_cache.reshape(N * S, D)
    v2 = v_cache.reshape(N * S, D)
    xk2 = xk.reshape(N, D)
    xv2 = xv.reshape(N, D)
    idxv = jnp.full((16,), jnp.asarray(layer_past_len, jnp.int32))
    mesh = plsc.VectorSubcoreMesh(
        core_axis_name="c", subcore_axis_name="s", num_cores=_NC, num_subcores=_NS
    )

    @functools.partial(
        pl.kernel,
        out_type=(
            jax.ShapeDtypeStruct((N * S, D), k_cache.dtype),
            jax.ShapeDtypeStruct((N * S, D), v_cache.dtype),
        ),
        mesh=mesh,
        scratch_types=[
            pltpu.VMEM((16,), jnp.int32),
            pltpu.VMEM((rows_per_core,), jnp.int32),
            pltpu.VMEM((rows_per_core, D), jnp.float32),
            pltpu.VMEM((rows_per_core, D), jnp.float32),
            pltpu.SemaphoreType.DMA,
            pltpu.SemaphoreType.DMA,
        ],
    )
    def sc_kernel(
        k_hbm, v_hbm, xk_hbm, xv_hbm, idx_hbm, ok_hbm, ov_hbm,
        idx_v, rowid_v, xkr_v, xvr_v, sem_b, sem_s,
    ):
        c = lax.axis_index("c")
        s = lax.axis_index("s")
        wid = c * _NS + s
        base_slab = wid * slabs_per_w
        copies = []
        for j in range(slabs_per_w):
            sl = pl.ds((base_slab + j) * S, S)
            copies.append(pltpu.make_async_copy(k_hbm.at[sl], ok_hbm.at[sl], sem_b))
            copies.append(pltpu.make_async_copy(v_hbm.at[sl], ov_hbm.at[sl], sem_b))
        for cp in copies:
            cp.start()
        for cp in copies:
            cp.wait()
        plsc.subcore_barrier()

        @pl.when(s == 0)
        def _scatter():
            pltpu.sync_copy(idx_hbm, idx_v)
            row0 = c * rows_per_core
            for j in range(rows_per_core // 16):
                rid = (lax.iota(jnp.int32, 16) + (row0 + j * 16)) * S + idx_v[...]
                rowid_v[pl.ds(j * 16, 16)] = rid
            pltpu.sync_copy(xk_hbm.at[pl.ds(row0, rows_per_core)], xkr_v)
            pltpu.sync_copy(xv_hbm.at[pl.ds(row0, rows_per_core)], xvr_v)
            sck = pltpu.async_copy(xkr_v, ok_hbm.at[rowid_v], sem_s)
            scv = pltpu.async_copy(xvr_v, ov_hbm.at[rowid_v], sem_s)
            sck.wait()
            scv.wait()

    ok, ov = sc_kernel(k2, v2, xk2, xv2, idxv)
    return ok.reshape(B, Hkv, S, D), ov.reshape(B, Hkv, S, D)


# speedup vs baseline: 38.0521x; 38.0521x over previous
import functools

import jax
import jax.numpy as jnp
from jax import lax
from jax.experimental import pallas as pl
from jax.experimental.pallas import tpu as pltpu
from jax.experimental.pallas import tpu_sc as plsc

_NC = 2
_NS = 16
_NW = _NC * _NS
_CH = 256


def kernel(k_cache, v_cache, xk, xv, layer_past_len):
    B, Hkv, S, D = k_cache.shape
    N = B * Hkv
    slabs_per_w = N // _NW
    rows_per_core = N // _NC
    k2 = k_cache.reshape(N * S, D)
    v2 = v_cache.reshape(N * S, D)
    xk2 = xk.reshape(N, D)
    xv2 = xv.reshape(N, D)
    idxv = jnp.full((16,), jnp.asarray(layer_past_len, jnp.int32))
    mesh = plsc.VectorSubcoreMesh(
        core_axis_name="c", subcore_axis_name="s", num_cores=_NC, num_subcores=_NS
    )

    @functools.partial(
        pl.kernel,
        out_type=(
            jax.ShapeDtypeStruct((N * S, D), k_cache.dtype),
            jax.ShapeDtypeStruct((N * S, D), v_cache.dtype),
        ),
        mesh=mesh,
        scratch_types=[
            pltpu.VMEM((16,), jnp.int32),
            pltpu.VMEM((rows_per_core,), jnp.int32),
            pltpu.VMEM((rows_per_core, D), jnp.float32),
            pltpu.VMEM((rows_per_core, D), jnp.float32),
            pltpu.VMEM((_CH, D), jnp.float32),
            pltpu.VMEM((_CH, D), jnp.float32),
            pltpu.SemaphoreType.DMA,
            pltpu.SemaphoreType.DMA,
            pltpu.SemaphoreType.DMA,
            pltpu.SemaphoreType.DMA,
            pltpu.SemaphoreType.DMA,
        ],
    )
    def sc_kernel(
        k_hbm, v_hbm, xk_hbm, xv_hbm, idx_hbm, ok_hbm, ov_hbm,
        idx_v, rowid_v, xkr_v, xvr_v, buf0, buf1,
        sin0, sin1, sout0, sout1, sem_s,
    ):
        c = lax.axis_index("c")
        s = lax.axis_index("s")
        wid = c * _NS + s
        base_slab = wid * slabs_per_w
        bufs = (buf0, buf1)
        sins = (sin0, sin1)
        souts = (sout0, sout1)
        chunks_per_slab = S // _CH
        chunk_list = []
        for j in range(slabs_per_w):
            for p in range(chunks_per_slab):
                off = (base_slab + j) * S + p * _CH
                chunk_list.append((k_hbm, ok_hbm, off))
                chunk_list.append((v_hbm, ov_hbm, off))
        total = len(chunk_list)
        incopies = [None, None]
        outcopies = [None, None]
        for t in range(total + 1):
            b = t % 2
            if t < total:
                if t >= 2:
                    outcopies[b].wait()
                src, _, off = chunk_list[t]
                cp = pltpu.make_async_copy(
                    src.at[pl.ds(off, _CH)], bufs[b], sins[b]
                )
                cp.start()
                incopies[b] = cp
            if t >= 1:
                p = (t - 1) % 2
                incopies[p].wait()
                _, dst, off = chunk_list[t - 1]
                ocp = pltpu.make_async_copy(
                    bufs[p], dst.at[pl.ds(off, _CH)], souts[p]
                )
                ocp.start()
                outcopies[p] = ocp
        outcopies[0].wait()
        outcopies[1].wait()
        plsc.subcore_barrier()

        @pl.when(s == 0)
        def _scatter():
            pltpu.sync_copy(idx_hbm, idx_v)
            row0 = c * rows_per_core
            for j in range(rows_per_core // 16):
                rid = (lax.iota(jnp.int32, 16) + (row0 + j * 16)) * S + idx_v[...]
                rowid_v[pl.ds(j * 16, 16)] = rid
            pltpu.sync_copy(xk_hbm.at[pl.ds(row0, rows_per_core)], xkr_v)
            pltpu.sync_copy(xv_hbm.at[pl.ds(row0, rows_per_core)], xvr_v)
            sck = pltpu.async_copy(xkr_v, ok_hbm.at[rowid_v], sem_s)
            scv = pltpu.async_copy(xvr_v, ov_hbm.at[rowid_v], sem_s)
            sck.wait()
            scv.wait()

    ok, ov = sc_kernel(k2, v2, xk2, xv2, idxv)
    return ok.reshape(B, Hkv, S, D), ov.reshape(B, Hkv, S, D)


# speedup vs baseline: 38.1185x; 1.0017x over previous
import functools

import jax
import jax.numpy as jnp
from jax import lax
from jax.experimental import pallas as pl
from jax.experimental.pallas import tpu as pltpu
from jax.experimental.pallas import tpu_sc as plsc

_NC = 2
_NS = 16
_NW = _NC * _NS
_CH = 256


def kernel(k_cache, v_cache, xk, xv, layer_past_len):
    B, Hkv, S, D = k_cache.shape
    N = B * Hkv
    slabs_per_w = N // _NW
    rows_per_core = N // _NC
    k2 = k_cache.reshape(N * S, D)
    v2 = v_cache.reshape(N * S, D)
    xk2 = xk.reshape(N, D)
    xv2 = xv.reshape(N, D)
    idxv = jnp.full((16,), jnp.asarray(layer_past_len, jnp.int32))
    mesh = plsc.VectorSubcoreMesh(
        core_axis_name="c", subcore_axis_name="s", num_cores=_NC, num_subcores=_NS
    )

    @functools.partial(
        pl.kernel,
        out_type=(
            jax.ShapeDtypeStruct((N * S, D), k_cache.dtype),
            jax.ShapeDtypeStruct((N * S, D), v_cache.dtype),
        ),
        mesh=mesh,
        scratch_types=[
            pltpu.VMEM((16,), jnp.int32),
            pltpu.VMEM((rows_per_core,), jnp.int32),
            pltpu.VMEM((_CH, D), jnp.float32),
            pltpu.VMEM((_CH, D), jnp.float32),
            pltpu.VMEM((_CH, D), jnp.float32),
            pltpu.SemaphoreType.DMA,
            pltpu.SemaphoreType.DMA,
            pltpu.SemaphoreType.DMA,
            pltpu.SemaphoreType.DMA,
            pltpu.SemaphoreType.DMA,
            pltpu.SemaphoreType.DMA,
            pltpu.SemaphoreType.DMA,
        ],
    )
    def sc_kernel(
        k_hbm, v_hbm, xk_hbm, xv_hbm, idx_hbm, ok_hbm, ov_hbm,
        idx_v, rowid_v, buf0, buf1, buf2,
        sin0, sin1, sin2, sout0, sout1, sout2, sem_s,
    ):
        c = lax.axis_index("c")
        s = lax.axis_index("s")
        wid = c * _NS + s
        base_slab = wid * slabs_per_w
        bufs = (buf0, buf1, buf2)
        sins = (sin0, sin1, sin2)
        souts = (sout0, sout1, sout2)
        nbuf = len(bufs)
        chunks_per_slab = S // _CH
        chunk_list = []
        for j in range(slabs_per_w):
            for p in range(chunks_per_slab):
                off = (base_slab + j) * S + p * _CH
                chunk_list.append((k_hbm, ok_hbm, off))
                chunk_list.append((v_hbm, ov_hbm, off))
        total = len(chunk_list)
        incopies = [None] * nbuf
        outcopies = [None] * nbuf
        for t in range(total + 1):
            if t < total:
                b = t % nbuf
                if t >= nbuf:
                    outcopies[b].wait()
                    outcopies[b] = None
                src, _, off = chunk_list[t]
                cp = pltpu.make_async_copy(
                    src.at[pl.ds(off, _CH)], bufs[b], sins[b]
                )
                cp.start()
                incopies[b] = cp
            if t >= 1:
                p = (t - 1) % nbuf
                incopies[p].wait()
                _, dst, off = chunk_list[t - 1]
                ocp = pltpu.make_async_copy(
                    bufs[p], dst.at[pl.ds(off, _CH)], souts[p]
                )
                ocp.start()
                outcopies[p] = ocp
        for p in range(nbuf):
            if outcopies[p] is not None:
                outcopies[p].wait()
        plsc.subcore_barrier()

        @pl.when(s == 0)
        def _scatter():
            pltpu.sync_copy(idx_hbm, idx_v)
            row0 = c * rows_per_core
            for j in range(rows_per_core // 16):
                rid = (lax.iota(jnp.int32, 16) + (row0 + j * 16)) * S + idx_v[...]
                rowid_v[pl.ds(j * 16, 16)] = rid
            xkr = buf0.at[pl.ds(0, rows_per_core)]
            xvr = buf1.at[pl.ds(0, rows_per_core)]
            pltpu.sync_copy(xk_hbm.at[pl.ds(row0, rows_per_core)], xkr)
            pltpu.sync_copy(xv_hbm.at[pl.ds(row0, rows_per_core)], xvr)
            sck = pltpu.async_copy(xkr, ok_hbm.at[rowid_v], sem_s)
            scv = pltpu.async_copy(xvr, ov_hbm.at[rowid_v], sem_s)
            sck.wait()
            scv.wait()

    ok, ov = sc_kernel(k2, v2, xk2, xv2, idxv)
    return ok.reshape(B, Hkv, S, D), ov.reshape(B, Hkv, S, D)


# speedup vs baseline: 42.0869x; 1.1041x over previous
import functools

import jax
import jax.numpy as jnp
from jax import lax
from jax.experimental import pallas as pl
from jax.experimental.pallas import tpu as pltpu
from jax.experimental.pallas import tpu_sc as plsc

_NC = 2
_NS = 16
_NW = _NC * _NS
_CH = 256
_G = 8


def _tc_body(idx_ref, c_ref, x_ref, o_ref):
    idx = idx_ref[0]
    o_ref[...] = c_ref[...]
    o_ref[:, pl.ds(idx, 1), :] = x_ref[...]


def _tc_update(cache3, x3, idx):
    N, S, D = cache3.shape
    cache_spec = pl.BlockSpec((_G, S, D), lambda i, idx_ref: (i, 0, 0))
    x_spec = pl.BlockSpec((_G, 1, D), lambda i, idx_ref: (i, 0, 0))
    grid_spec = pltpu.PrefetchScalarGridSpec(
        num_scalar_prefetch=1,
        grid=(N // _G,),
        in_specs=[cache_spec, x_spec],
        out_specs=cache_spec,
    )
    return pl.pallas_call(
        _tc_body,
        grid_spec=grid_spec,
        out_shape=jax.ShapeDtypeStruct(cache3.shape, cache3.dtype),
    )(idx, cache3, x3)


def _sc_update(cache2, x2, idxv, N, S, D):
    slabs_per_w = N // _NW
    rows_per_core = N // _NC
    mesh = plsc.VectorSubcoreMesh(
        core_axis_name="c", subcore_axis_name="s", num_cores=_NC, num_subcores=_NS
    )

    @functools.partial(
        pl.kernel,
        out_type=jax.ShapeDtypeStruct((N * S, D), cache2.dtype),
        mesh=mesh,
        scratch_types=[
            pltpu.VMEM((16,), jnp.int32),
            pltpu.VMEM((rows_per_core,), jnp.int32),
            pltpu.VMEM((_CH, D), jnp.float32),
            pltpu.VMEM((_CH, D), jnp.float32),
            pltpu.VMEM((_CH, D), jnp.float32),
            pltpu.SemaphoreType.DMA,
            pltpu.SemaphoreType.DMA,
            pltpu.SemaphoreType.DMA,
            pltpu.SemaphoreType.DMA,
            pltpu.SemaphoreType.DMA,
            pltpu.SemaphoreType.DMA,
            pltpu.SemaphoreType.DMA,
        ],
    )
    def sc_kernel(
        c_hbm, x_hbm, idx_hbm, o_hbm,
        idx_v, rowid_v, buf0, buf1, buf2,
        sin0, sin1, sin2, sout0, sout1, sout2, sem_s,
    ):
        c = lax.axis_index("c")
        s = lax.axis_index("s")
        wid = c * _NS + s
        base_slab = wid * slabs_per_w
        bufs = (buf0, buf1, buf2)
        sins = (sin0, sin1, sin2)
        souts = (sout0, sout1, sout2)
        nbuf = len(bufs)
        chunks_per_slab = S // _CH
        chunk_list = []
        for j in range(slabs_per_w):
            for p in range(chunks_per_slab):
                chunk_list.append((base_slab + j) * S + p * _CH)
        total = len(chunk_list)
        incopies = [None] * nbuf
        outcopies = [None] * nbuf
        for t in range(total + 1):
            if t < total:
                b = t % nbuf
                if t >= nbuf:
                    outcopies[b].wait()
                    outcopies[b] = None
                cp = pltpu.make_async_copy(
                    c_hbm.at[pl.ds(chunk_list[t], _CH)], bufs[b], sins[b]
                )
                cp.start()
                incopies[b] = cp
            if t >= 1:
                p = (t - 1) % nbuf
                incopies[p].wait()
                ocp = pltpu.make_async_copy(
                    bufs[p], o_hbm.at[pl.ds(chunk_list[t - 1], _CH)], souts[p]
                )
                ocp.start()
                outcopies[p] = ocp
        for p in range(nbuf):
            if outcopies[p] is not None:
                outcopies[p].wait()
        plsc.subcore_barrier()

        @pl.when(s == 0)
        def _scatter():
            pltpu.sync_copy(idx_hbm, idx_v)
            row0 = c * rows_per_core
            for j in range(rows_per_core // 16):
                rid = (lax.iota(jnp.int32, 16) + (row0 + j * 16)) * S + idx_v[...]
                rowid_v[pl.ds(j * 16, 16)] = rid
            xr = buf0.at[pl.ds(0, rows_per_core)]
            pltpu.sync_copy(x_hbm.at[pl.ds(row0, rows_per_core)], xr)
            sc = pltpu.async_copy(xr, o_hbm.at[rowid_v], sem_s)
            sc.wait()

    return sc_kernel(cache2, x2, idxv)


def kernel(k_cache, v_cache, xk, xv, layer_past_len):
    B, Hkv, S, D = k_cache.shape
    N = B * Hkv
    idx = jnp.asarray(layer_past_len, jnp.int32).reshape((1,))
    idxv = jnp.full((16,), jnp.asarray(layer_past_len, jnp.int32))
    ov = _sc_update(v_cache.reshape(N * S, D), xv.reshape(N, D), idxv, N, S, D)
    ok = _tc_update(k_cache.reshape(N, S, D), xk.reshape(N, 1, D), idx)
    return ok.reshape(B, Hkv, S, D), ov.reshape(B, Hkv, S, D)
